# baseline (speedup 1.0000x reference)
import functools

import jax
import jax.numpy as jnp
from jax import lax
from jax.experimental import pallas as pl
from jax.experimental.pallas import tpu as pltpu

B = 1024
T = 8
D = 64
N = 100000
MAX_K = 60
KPAD = 64
TOP_K = 8
CDIM = 17
CHUNK = 4096
NCHUNK = (N + CHUNK - 1) // CHUNK
NPAD = NCHUNK * CHUNK
GROUPS = CHUNK // 128
INF = 3.0e38


def _knn_kernel(hs_ref, keys_ref, out_d_ref, out_i_ref, q_ref, qn_ref,
                run_d_ref, run_i_ref, dm_ref, gmin_ref):
    i = pl.program_id(0)

    @pl.when(i == 0)
    def _init():
        acc = jnp.zeros((B, D), jnp.float32)
        for t in range(T):
            acc = acc + hs_ref[:, t * D:(t + 1) * D]
        q = acc * (1.0 / T)
        q_ref[...] = q
        qn_ref[...] = jnp.sum(q * q, axis=1, keepdims=True)
        run_d_ref[...] = jnp.full((B, KPAD), INF, jnp.float32)
        run_i_ref[...] = jnp.zeros((B, KPAD), jnp.float32)

    q = q_ref[...]
    kt = keys_ref[...]
    qk = jnp.dot(q, kt, preferred_element_type=jnp.float32)
    kn = jnp.sum(kt * kt, axis=0, keepdims=True)
    d = qn_ref[...] - 2.0 * qk + kn
    ci = lax.broadcasted_iota(jnp.int32, (1, CHUNK), 1)
    d = jnp.where(ci < N - i * CHUNK, d, INF)
    dm_ref[...] = d

    l32 = lax.broadcasted_iota(jnp.int32, (1, GROUPS), 1)
    acc = jnp.full((B, GROUPS), INF, jnp.float32)
    for g in range(GROUPS):
        gm = jnp.min(d[:, g * 128:(g + 1) * 128], axis=1, keepdims=True)
        acc = jnp.where(l32 == g, gm, acc)
    gmin_ref[...] = acc

    lane = lax.broadcasted_iota(jnp.int32, (1, KPAD), 1).astype(jnp.float32)
    li = lax.broadcasted_iota(jnp.int32, (1, 128), 1).astype(jnp.float32)
    base = (i * CHUNK).astype(jnp.float32)

    def merge_round(more):
        del more
        for g in range(GROUPS):
            thr = run_d_ref[:, KPAD - 1:KPAD]
            fire = jnp.any(gmin_ref[:, g:g + 1] < thr)

            @pl.when(fire)
            def _(g=g):
                sl = dm_ref[:, g * 128:(g + 1) * 128]
                rd = run_d_ref[...]
                ri = run_i_ref[...]
                v1 = gmin_ref[:, g:g + 1]
                pos1 = jnp.min(jnp.where(sl == v1, li, 10000.0),
                               axis=1, keepdims=True)
                sl = jnp.where(li == pos1, INF, sl)
                v2 = jnp.min(sl, axis=1, keepdims=True)
                pos2 = jnp.min(jnp.where(sl == v2, li, 10000.0),
                               axis=1, keepdims=True)
                sl = jnp.where(li == pos2, INF, sl)
                dm_ref[:, g * 128:(g + 1) * 128] = sl
                gmin_ref[:, g:g + 1] = jnp.min(sl, axis=1, keepdims=True)
                i1 = base + (g * 128) + pos1
                i2 = base + (g * 128) + pos2
                p1 = jnp.sum((rd <= v1).astype(jnp.float32),
                             axis=1, keepdims=True)
                p2 = jnp.sum((rd <= v2).astype(jnp.float32),
                             axis=1, keepdims=True) + 1.0
                r1d = pltpu.roll(rd, 1, 1)
                r2d = pltpu.roll(rd, 2, 1)
                r1i = pltpu.roll(ri, 1, 1)
                r2i = pltpu.roll(ri, 2, 1)
                nd = jnp.where(lane < p1, rd,
                               jnp.where(lane == p1, v1,
                                         jnp.where(lane < p2, r1d,
                                                   jnp.where(lane == p2, v2,
                                                             r2d))))
                ni = jnp.where(lane < p1, ri,
                               jnp.where(lane == p1, i1,
                                         jnp.where(lane < p2, r1i,
                                                   jnp.where(lane == p2, i2,
                                                             r2i))))
                run_d_ref[...] = nd
                run_i_ref[...] = ni
        thr = run_d_ref[:, KPAD - 1:KPAD]
        return jnp.any(gmin_ref[...] < thr)

    for _ in range(3):
        merge_round(True)

    @pl.when(i < 3)
    def _warmup():
        for _ in range(5):
            merge_round(True)

    more0 = jnp.any(gmin_ref[...] < run_d_ref[:, KPAD - 1:KPAD])
    lax.while_loop(lambda m: m, merge_round, more0)

    @pl.when(i == NCHUNK - 1)
    def _out():
        out_d_ref[...] = run_d_ref[...]
        out_i_ref[...] = run_i_ref[...].astype(jnp.int32)


def _knn_topk(hs2d, keys_pad):
    return pl.pallas_call(
        _knn_kernel,
        grid=(NCHUNK,),
        in_specs=[
            pl.BlockSpec((B, T * D), lambda i: (0, 0)),
            pl.BlockSpec((D, CHUNK), lambda i: (0, i)),
        ],
        out_specs=[
            pl.BlockSpec((B, KPAD), lambda i: (0, 0)),
            pl.BlockSpec((B, KPAD), lambda i: (0, 0)),
        ],
        out_shape=[
            jax.ShapeDtypeStruct((B, KPAD), jnp.float32),
            jax.ShapeDtypeStruct((B, KPAD), jnp.int32),
        ],
        scratch_shapes=[
            pltpu.VMEM((B, D), jnp.float32),
            pltpu.VMEM((B, 1), jnp.float32),
            pltpu.VMEM((B, KPAD), jnp.float32),
            pltpu.VMEM((B, KPAD), jnp.float32),
            pltpu.VMEM((B, CHUNK), jnp.float32),
            pltpu.VMEM((B, GROUPS), jnp.float32),
        ],
    )(hs2d, keys_pad)


def _head_kernel(d_ref, sc_ref, p_ref, conf_ref,
                 k1wT_ref, k1b_ref, k2wT_ref, k2b_ref,
                 lk1wT_ref, lk1b_ref, lk2w_ref, lk2b_ref,
                 lw1dT_ref, lw1sr_ref, lw1sp_ref, lw1t8_ref, lw1b_ref,
                 lw2w_ref, lw2b_ref,
                 final_ref, np_ref, lam0_ref, lam1_ref):
    d = d_ref[...]
    sc = sc_ref[...]
    p = p_ref[...]
    lane64 = lax.broadcasted_iota(jnp.int32, (1, KPAD), 1)

    dot = functools.partial(jnp.dot, preferred_element_type=jnp.float32)
    h1 = jnp.tanh(dot(d, k1wT_ref[...]) + k1b_ref[0:1, :])
    logits = dot(h1, k2wT_ref[...]) + k2b_ref[0:1, :]
    logits = jnp.where(lane64 < MAX_K, logits, -INF)
    mx = jnp.max(logits, axis=1, keepdims=True)
    e = jnp.exp(logits - mx)
    w = e / jnp.sum(e, axis=1, keepdims=True)
    np_s = jnp.sum(w * sc, axis=1, keepdims=True)

    h2 = jnp.tanh(dot(d, lk1wT_ref[...]) + lk1b_ref[0:1, :])
    knn_lam = (jnp.sum(h2 * lk2w_ref[0:1, :], axis=1, keepdims=True)
               + lk2b_ref[0:1, 0:1])

    acc = jnp.zeros((B, CDIM), jnp.float32)
    for t in range(T):
        acc = acc + conf_ref[:, t * CDIM:(t + 1) * CDIM]
    conf = acc * (1.0 / T)
    i17 = lax.broadcasted_iota(jnp.int32, (1, CDIM), 1)

    def bin_idx(x):
        xi = jnp.clip((x - 1.0) * 4.0, 0.0, 16.0)
        return xi.astype(jnp.int32)

    sr = bin_idx(p)
    sp = bin_idx(np_s)
    sr_conf = jnp.sum(jnp.where(i17 == sr, conf, 0.0), axis=1, keepdims=True)
    sp_conf = jnp.sum(jnp.where(i17 == sp, conf, 0.0), axis=1, keepdims=True)

    pre = (dot(d, lw1dT_ref[...])
           + sr_conf * lw1sr_ref[0:1, :]
           + sp_conf * lw1sp_ref[0:1, :]
           + lw1b_ref[0:1, :])
    c = conf
    for j in range(TOP_K):
        mxc = jnp.max(c, axis=1, keepdims=True)
        pos = jnp.min(jnp.where(c == mxc, i17, 99), axis=1, keepdims=True)
        pre = pre + mxc * lw1t8_ref[j:j + 1, :]
        c = jnp.where(i17 == pos, -INF, c)
    h3 = jnp.tanh(pre)
    wav_lam = (jnp.sum(h3 * lw2w_ref[0:1, :], axis=1, keepdims=True)
               + lw2b_ref[0:1, 0:1])

    m2 = jnp.maximum(knn_lam, wav_lam)
    e0 = jnp.exp(knn_lam - m2)
    e1 = jnp.exp(wav_lam - m2)
    s2 = e0 + e1
    lam0 = e0 / s2
    lam1 = e1 / s2
    final_ref[...] = lam0 * np_s + lam1 * p
    np_ref[...] = np_s
    lam0_ref[...] = lam0
    lam1_ref[...] = lam1


def _heads(knn_d, knn_sc, p2d, conf2d, *weights):
    return pl.pallas_call(
        _head_kernel,
        in_specs=[pl.BlockSpec(a.shape, lambda: (0, 0)) for a in
                  (knn_d, knn_sc, p2d, conf2d) + weights],
        out_specs=[pl.BlockSpec((B, 1), lambda: (0, 0))] * 4,
        out_shape=[jax.ShapeDtypeStruct((B, 1), jnp.float32)] * 4,
    )(knn_d, knn_sc, p2d, conf2d, *weights)


def kernel(p_scores, hs, confidences, datastore_keys, datastore_scores,
           k1w, k1b, k2w, k2b, lk1w, lk1b, lk2w, lk2b, lw1w, lw1b, lw2w, lw2b):
    hs2d = hs.reshape(B, T * D)
    keys_pad = jnp.pad(datastore_keys.T, ((0, 0), (0, NPAD - N)))
    knn_d, knn_i = _knn_topk(hs2d, keys_pad)

    knn_sc = jnp.take(datastore_scores, knn_i.reshape(-1), axis=0)
    knn_sc = knn_sc.reshape(B, KPAD)

    def tile8(v):
        return jnp.tile(v.reshape(1, -1), (8, 1)).astype(jnp.float32)

    k1wT = jnp.pad(k1w.T, ((0, KPAD - MAX_K), (0, 0)))
    k2wT = jnp.pad(k2w.T, ((0, 0), (0, KPAD - MAX_K)))
    k2b_p = jnp.pad(k2b, (0, KPAD - MAX_K))
    lk1wT = jnp.pad(lk1w.T, ((0, KPAD - MAX_K), (0, 0)))
    lw1dT = jnp.pad(lw1w[:, 10:].T, ((0, KPAD - MAX_K), (0, 0)))
    lw1t8 = lw1w[:, 2:10].T

    weights = (k1wT, tile8(k1b), k2wT, tile8(k2b_p),
               lk1wT, tile8(lk1b), tile8(lk2w), tile8(lk2b),
               lw1dT, tile8(lw1w[:, 0]), tile8(lw1w[:, 1]), lw1t8,
               tile8(lw1b), tile8(lw2w), tile8(lw2b))

    p2d = p_scores.reshape(B, 1)
    conf2d = confidences.reshape(B, T * CDIM)
    final, np_s, lam0, lam1 = _heads(knn_d, knn_sc, p2d, conf2d, *weights)
    return (final, p_scores, np_s[:, 0], lam0[:, 0], lam1[:, 0])


# speedup vs baseline: 1.6211x; 1.6211x over previous
import functools

import jax
import jax.numpy as jnp
from jax import lax
from jax.experimental import pallas as pl
from jax.experimental.pallas import tpu as pltpu

B = 1024
T = 8
D = 64
N = 100000
MAX_K = 60
KPAD = 64
TOP_K = 8
CDIM = 17
CHUNK = 4096
NCHUNK = (N + CHUNK - 1) // CHUNK
NPAD = NCHUNK * CHUNK
GROUPS = CHUNK // 128
INF = 3.0e38


def _knn_kernel(hs_ref, keys_ref, out_d_ref, out_i_ref, q_ref, qn_ref,
                run_d_ref, run_i_ref, dm_ref):
    i = pl.program_id(0)

    @pl.when(i == 0)
    def _init():
        acc = jnp.zeros((B, D), jnp.float32)
        for t in range(T):
            acc = acc + hs_ref[:, t * D:(t + 1) * D]
        q = acc * (1.0 / T)
        q_ref[...] = q
        qn_ref[...] = jnp.sum(q * q, axis=1, keepdims=True)
        run_d_ref[...] = jnp.full((B, KPAD), INF, jnp.float32)
        run_i_ref[...] = jnp.zeros((B, KPAD), jnp.float32)

    q = q_ref[...]
    kt = keys_ref[...]
    qk = jnp.dot(q, kt, preferred_element_type=jnp.float32)
    kn = jnp.sum(kt * kt, axis=0, keepdims=True)
    d = qn_ref[...] - 2.0 * qk + kn
    ci = lax.broadcasted_iota(jnp.int32, (1, CHUNK), 1)
    d = jnp.where(ci < N - i * CHUNK, d, INF)
    dm_ref[...] = d

    lane = lax.broadcasted_iota(jnp.int32, (1, KPAD), 1).astype(jnp.float32)
    li = lax.broadcasted_iota(jnp.int32, (1, 128), 1).astype(jnp.float32)
    base = (i * CHUNK).astype(jnp.float32)

    def merge_round(more):
        del more
        for g in range(GROUPS):
            sl0 = dm_ref[:, g * 128:(g + 1) * 128]
            thr = run_d_ref[:, KPAD - 1:KPAD]
            fire = jnp.any(sl0 < thr)

            @pl.when(fire)
            def _(sl=sl0, g=g):
                rd = run_d_ref[...]
                ri = run_i_ref[...]
                v1 = jnp.min(sl, axis=1, keepdims=True)
                pos1 = jnp.min(jnp.where(sl == v1, li, 10000.0),
                               axis=1, keepdims=True)
                sl = jnp.where(li == pos1, INF, sl)
                v2 = jnp.min(sl, axis=1, keepdims=True)
                pos2 = jnp.min(jnp.where(sl == v2, li, 10000.0),
                               axis=1, keepdims=True)
                sl = jnp.where(li == pos2, INF, sl)
                dm_ref[:, g * 128:(g + 1) * 128] = sl
                i1 = base + (g * 128) + pos1
                i2 = base + (g * 128) + pos2
                p1 = jnp.sum((rd <= v1).astype(jnp.float32),
                             axis=1, keepdims=True)
                p2 = jnp.sum((rd <= v2).astype(jnp.float32),
                             axis=1, keepdims=True) + 1.0
                r1d = pltpu.roll(rd, 1, 1)
                r2d = pltpu.roll(rd, 2, 1)
                r1i = pltpu.roll(ri, 1, 1)
                r2i = pltpu.roll(ri, 2, 1)
                nd = jnp.where(lane < p1, rd,
                               jnp.where(lane == p1, v1,
                                         jnp.where(lane < p2, r1d,
                                                   jnp.where(lane == p2, v2,
                                                             r2d))))
                ni = jnp.where(lane < p1, ri,
                               jnp.where(lane == p1, i1,
                                         jnp.where(lane < p2, r1i,
                                                   jnp.where(lane == p2, i2,
                                                             r2i))))
                run_d_ref[...] = nd
                run_i_ref[...] = ni
        thr = run_d_ref[:, KPAD - 1:KPAD]
        return jnp.any(dm_ref[...] < thr)

    more0 = jnp.any(d < run_d_ref[:, KPAD - 1:KPAD])
    lax.while_loop(lambda m: m, merge_round, more0)

    @pl.when(i == NCHUNK - 1)
    def _out():
        out_d_ref[...] = run_d_ref[...]
        out_i_ref[...] = run_i_ref[...].astype(jnp.int32)


def _knn_topk(hs2d, keys_pad):
    return pl.pallas_call(
        _knn_kernel,
        grid=(NCHUNK,),
        in_specs=[
            pl.BlockSpec((B, T * D), lambda i: (0, 0)),
            pl.BlockSpec((D, CHUNK), lambda i: (0, i)),
        ],
        out_specs=[
            pl.BlockSpec((B, KPAD), lambda i: (0, 0)),
            pl.BlockSpec((B, KPAD), lambda i: (0, 0)),
        ],
        out_shape=[
            jax.ShapeDtypeStruct((B, KPAD), jnp.float32),
            jax.ShapeDtypeStruct((B, KPAD), jnp.int32),
        ],
        scratch_shapes=[
            pltpu.VMEM((B, D), jnp.float32),
            pltpu.VMEM((B, 1), jnp.float32),
            pltpu.VMEM((B, KPAD), jnp.float32),
            pltpu.VMEM((B, KPAD), jnp.float32),
            pltpu.VMEM((B, CHUNK), jnp.float32),
        ],
    )(hs2d, keys_pad)


def _head_kernel(d_ref, sc_ref, p_ref, conf_ref,
                 k1wT_ref, k1b_ref, k2wT_ref, k2b_ref,
                 lk1wT_ref, lk1b_ref, lk2w_ref, lk2b_ref,
                 lw1dT_ref, lw1sr_ref, lw1sp_ref, lw1t8_ref, lw1b_ref,
                 lw2w_ref, lw2b_ref,
                 final_ref, np_ref, lam0_ref, lam1_ref):
    d = d_ref[...]
    sc = sc_ref[...]
    p = p_ref[...]
    lane64 = lax.broadcasted_iota(jnp.int32, (1, KPAD), 1)

    dot = functools.partial(jnp.dot, preferred_element_type=jnp.float32)
    h1 = jnp.tanh(dot(d, k1wT_ref[...]) + k1b_ref[0:1, :])
    logits = dot(h1, k2wT_ref[...]) + k2b_ref[0:1, :]
    logits = jnp.where(lane64 < MAX_K, logits, -INF)
    mx = jnp.max(logits, axis=1, keepdims=True)
    e = jnp.exp(logits - mx)
    w = e / jnp.sum(e, axis=1, keepdims=True)
    np_s = jnp.sum(w * sc, axis=1, keepdims=True)

    h2 = jnp.tanh(dot(d, lk1wT_ref[...]) + lk1b_ref[0:1, :])
    knn_lam = (jnp.sum(h2 * lk2w_ref[0:1, :], axis=1, keepdims=True)
               + lk2b_ref[0:1, 0:1])

    acc = jnp.zeros((B, CDIM), jnp.float32)
    for t in range(T):
        acc = acc + conf_ref[:, t * CDIM:(t + 1) * CDIM]
    conf = acc * (1.0 / T)
    i17 = lax.broadcasted_iota(jnp.int32, (1, CDIM), 1)

    def bin_idx(x):
        xi = jnp.clip((x - 1.0) * 4.0, 0.0, 16.0)
        return xi.astype(jnp.int32)

    sr = bin_idx(p)
    sp = bin_idx(np_s)
    sr_conf = jnp.sum(jnp.where(i17 == sr, conf, 0.0), axis=1, keepdims=True)
    sp_conf = jnp.sum(jnp.where(i17 == sp, conf, 0.0), axis=1, keepdims=True)

    pre = (dot(d, lw1dT_ref[...])
           + sr_conf * lw1sr_ref[0:1, :]
           + sp_conf * lw1sp_ref[0:1, :]
           + lw1b_ref[0:1, :])
    c = conf
    for j in range(TOP_K):
        mxc = jnp.max(c, axis=1, keepdims=True)
        pos = jnp.min(jnp.where(c == mxc, i17, 99), axis=1, keepdims=True)
        pre = pre + mxc * lw1t8_ref[j:j + 1, :]
        c = jnp.where(i17 == pos, -INF, c)
    h3 = jnp.tanh(pre)
    wav_lam = (jnp.sum(h3 * lw2w_ref[0:1, :], axis=1, keepdims=True)
               + lw2b_ref[0:1, 0:1])

    m2 = jnp.maximum(knn_lam, wav_lam)
    e0 = jnp.exp(knn_lam - m2)
    e1 = jnp.exp(wav_lam - m2)
    s2 = e0 + e1
    lam0 = e0 / s2
    lam1 = e1 / s2
    final_ref[...] = lam0 * np_s + lam1 * p
    np_ref[...] = np_s
    lam0_ref[...] = lam0
    lam1_ref[...] = lam1


def _heads(knn_d, knn_sc, p2d, conf2d, *weights):
    return pl.pallas_call(
        _head_kernel,
        in_specs=[pl.BlockSpec(a.shape, lambda: (0, 0)) for a in
                  (knn_d, knn_sc, p2d, conf2d) + weights],
        out_specs=[pl.BlockSpec((B, 1), lambda: (0, 0))] * 4,
        out_shape=[jax.ShapeDtypeStruct((B, 1), jnp.float32)] * 4,
    )(knn_d, knn_sc, p2d, conf2d, *weights)


def kernel(p_scores, hs, confidences, datastore_keys, datastore_scores,
           k1w, k1b, k2w, k2b, lk1w, lk1b, lk2w, lk2b, lw1w, lw1b, lw2w, lw2b):
    hs2d = hs.reshape(B, T * D)
    keys_pad = jnp.pad(datastore_keys.T, ((0, 0), (0, NPAD - N)))
    knn_d, knn_i = _knn_topk(hs2d, keys_pad)

    knn_sc = jnp.take(datastore_scores, knn_i.reshape(-1), axis=0)
    knn_sc = knn_sc.reshape(B, KPAD)

    def tile8(v):
        return jnp.tile(v.reshape(1, -1), (8, 1)).astype(jnp.float32)

    k1wT = jnp.pad(k1w.T, ((0, KPAD - MAX_K), (0, 0)))
    k2wT = jnp.pad(k2w.T, ((0, 0), (0, KPAD - MAX_K)))
    k2b_p = jnp.pad(k2b, (0, KPAD - MAX_K))
    lk1wT = jnp.pad(lk1w.T, ((0, KPAD - MAX_K), (0, 0)))
    lw1dT = jnp.pad(lw1w[:, 10:].T, ((0, KPAD - MAX_K), (0, 0)))
    lw1t8 = lw1w[:, 2:10].T

    weights = (k1wT, tile8(k1b), k2wT, tile8(k2b_p),
               lk1wT, tile8(lk1b), tile8(lk2w), tile8(lk2b),
               lw1dT, tile8(lw1w[:, 0]), tile8(lw1w[:, 1]), lw1t8,
               tile8(lw1b), tile8(lw2w), tile8(lw2b))

    p2d = p_scores.reshape(B, 1)
    conf2d = confidences.reshape(B, T * CDIM)
    final, np_s, lam0, lam1 = _heads(knn_d, knn_sc, p2d, conf2d, *weights)
    return (final, p_scores, np_s[:, 0], lam0[:, 0], lam1[:, 0])
